# Initial kernel scaffold; baseline (speedup 1.0000x reference)
import jax
import jax.numpy as jnp
from jax.experimental import pallas as pl


def kernel(x, edge_index, W1, b1, W2, b2):
    raise NotImplementedError("write your pallas kernel here")


# speedup vs baseline: 31.5887x; 31.5887x over previous
import functools

import jax
import jax.numpy as jnp
from jax import lax
from jax.experimental import pallas as pl
from jax.experimental.pallas import tpu as pltpu
from jax.experimental.pallas import tpu_sc as plsc

N = 10000
E = 320000
F = 128
H = 4
C = 16

D = 16
NW = 32
CH = 128
NCH = -(-E // (NW * CH))
EPW = NCH * CH
EPAD = NW * EPW
NGARB = N
ZR = 632
NP = ZR * 16

_mesh = plsc.VectorSubcoreMesh(core_axis_name="c", subcore_axis_name="s")
_sc_params = pltpu.CompilerParams(use_tc_tiling_on_sc=False)


def _sc_common(cid, sid, out_hbm, zb, acc, edge_loop):
    def zrow(i, c):
        zb[i, :] = jnp.zeros((D,), jnp.float32)
        return c

    lax.fori_loop(0, ZR, zrow, 0)
    pltpu.sync_copy(zb, acc.at[pl.ds(sid * ZR, ZR)])
    plsc.subcore_barrier()

    edge_loop()

    plsc.subcore_barrier()
    pltpu.sync_copy(acc.at[pl.ds(sid * ZR, ZR)], zb)
    pltpu.sync_copy(zb, out_hbm.at[cid, pl.ds(sid * ZR, ZR)])


@functools.partial(
    pl.kernel,
    out_type=jax.ShapeDtypeStruct((2, NP, D), jnp.float32),
    mesh=_mesh,
    compiler_params=_sc_params,
    scratch_types=[
        pltpu.VMEM((NCH, CH), jnp.int32),
        pltpu.VMEM((CH, D), jnp.float32),
        pltpu.VMEM((ZR, D), jnp.float32),
        pltpu.VMEM_SHARED((NP, D), jnp.float32),
    ],
)
def _sc_degree(dst_hbm, out_hbm, dstv, rows, zb, acc):
    cid = lax.axis_index("c")
    sid = lax.axis_index("s")
    wid = sid * 2 + cid

    def edge_loop():
        pltpu.sync_copy(dst_hbm.at[wid], dstv)

        def orow(i, c):
            rows[i, :] = jnp.ones((D,), jnp.float32)
            return c

        lax.fori_loop(0, CH, orow, 0)

        def step(j, c):
            pltpu.sync_copy(rows, acc.at[dstv.at[j]], add=True)
            return c

        lax.fori_loop(0, NCH, step, 0)

    _sc_common(cid, sid, out_hbm, zb, acc, edge_loop)


@functools.partial(
    pl.kernel,
    out_type=jax.ShapeDtypeStruct((2, NP, D), jnp.float32),
    mesh=_mesh,
    compiler_params=_sc_params,
    scratch_types=[
        pltpu.VMEM((NCH, CH), jnp.int32),
        pltpu.VMEM((NCH, CH), jnp.int32),
        pltpu.VMEM((CH, D), jnp.float32),
        pltpu.VMEM((ZR, D), jnp.float32),
        pltpu.VMEM_SHARED((NP, D), jnp.float32),
        pltpu.SemaphoreType.DMA,
    ],
)
def _sc_aggregate(src_hbm, dst_hbm, ytab_hbm, out_hbm, srcv, dstv, rows, zb,
                  acc, sem):
    cid = lax.axis_index("c")
    sid = lax.axis_index("s")
    wid = sid * 2 + cid

    def edge_loop():
        pltpu.sync_copy(src_hbm.at[wid], srcv)
        pltpu.sync_copy(dst_hbm.at[wid], dstv)

        def step(j, c):
            pltpu.async_copy(ytab_hbm.at[srcv.at[j]], rows, sem).wait()
            pltpu.sync_copy(rows, acc.at[dstv.at[j]], add=True)
            return c

        lax.fori_loop(0, NCH, step, 0)

    _sc_common(cid, sid, out_hbm, zb, acc, edge_loop)


def _k1_body(x_ref, w1_ref, degp_ref, y1_ref, dis_ref):
    deg = degp_ref[0] + degp_ref[1] + 1.0
    dis = lax.rsqrt(deg)
    dis_ref[...] = dis
    xw = jnp.dot(x_ref[...], w1_ref[...], preferred_element_type=jnp.float32)
    y1_ref[: N, :] = xw * dis[: N, :]
    y1_ref[N:, :] = jnp.zeros((NP - N, D), jnp.float32)


_k1 = pl.pallas_call(
    _k1_body,
    out_shape=[
        jax.ShapeDtypeStruct((NP, D), jnp.float32),
        jax.ShapeDtypeStruct((NP, D), jnp.float32),
    ],
)


def _k2_body(y1_ref, dis_ref, aggp_ref, w2_ref, b1p_ref, y2_ref):
    agg = aggp_ref[0] + aggp_ref[1]
    h = jnp.maximum(dis_ref[...] * (agg + y1_ref[...]) + b1p_ref[...], 0.0)
    hw2 = jnp.dot(h, w2_ref[...], preferred_element_type=jnp.float32)
    y2_ref[...] = dis_ref[...] * hw2


_k2 = pl.pallas_call(
    _k2_body,
    out_shape=jax.ShapeDtypeStruct((NP, D), jnp.float32),
)


def _k3_body(y2_ref, dis_ref, aggp_ref, b2_ref, logits_ref, probs_ref):
    agg = aggp_ref[0, : N, :] + aggp_ref[1, : N, :]
    lg = dis_ref[: N, :] * (agg + y2_ref[: N, :]) + b2_ref[...]
    logits_ref[...] = lg
    m = jnp.max(lg, axis=1, keepdims=True)
    e = jnp.exp(lg - m)
    probs_ref[...] = e / jnp.sum(e, axis=1, keepdims=True)


_k3 = pl.pallas_call(
    _k3_body,
    out_shape=[
        jax.ShapeDtypeStruct((N, C), jnp.float32),
        jax.ShapeDtypeStruct((N, C), jnp.float32),
    ],
)


def kernel(x, edge_index, W1, b1, W2, b2):
    src = edge_index[0]
    dst = edge_index[1]
    pad = EPAD - E
    srcp = jnp.concatenate([src, jnp.zeros((pad,), jnp.int32)])
    srcp = srcp.reshape(NW, NCH, CH)
    dstp = jnp.concatenate([dst, jnp.full((pad,), NGARB, jnp.int32)])
    dstp = dstp.reshape(NW, NCH, CH)

    W1p = jnp.pad(W1, ((0, 0), (0, D - H)))
    b1p = jnp.pad(b1, (0, D - H))
    W2p = jnp.pad(W2, ((0, D - H), (0, 0)))

    degp = _sc_degree(dstp)
    y1, dis = _k1(x, W1p, degp)
    agg1 = _sc_aggregate(srcp, dstp, y1)
    y2 = _k2(y1, dis, agg1, W2p, b1p)
    agg2 = _sc_aggregate(srcp, dstp, y2)
    logits, probs = _k3(y2, dis, agg2, b2)
    return logits, probs


# speedup vs baseline: 36.1441x; 1.1442x over previous
import functools

import jax
import jax.numpy as jnp
from jax import lax
from jax.experimental import pallas as pl
from jax.experimental.pallas import tpu as pltpu
from jax.experimental.pallas import tpu_sc as plsc

N = 10000
E = 320000
F = 128
H = 4
C = 16

D = 16
NW = 32
CH = 128
NCH = 2 * (-(-E // (NW * CH * 2)))
EPW = NCH * CH
EPAD = NW * EPW
NGARB = N
ZR = 632
NP = ZR * 16

_mesh = plsc.VectorSubcoreMesh(core_axis_name="c", subcore_axis_name="s")
_sc_params = pltpu.CompilerParams(use_tc_tiling_on_sc=False)


def _sc_common(cid, sid, out_hbm, zb, acc, edge_loop):
    def zrow(i, c):
        zb[i, :] = jnp.zeros((D,), jnp.float32)
        return c

    lax.fori_loop(0, ZR, zrow, 0)
    pltpu.sync_copy(zb, acc.at[pl.ds(sid * ZR, ZR)])
    plsc.subcore_barrier()

    edge_loop()

    plsc.subcore_barrier()
    pltpu.sync_copy(acc.at[pl.ds(sid * ZR, ZR)], zb)
    pltpu.sync_copy(zb, out_hbm.at[cid, pl.ds(sid * ZR, ZR)])


@functools.partial(
    pl.kernel,
    out_type=jax.ShapeDtypeStruct((2, NP, D), jnp.float32),
    mesh=_mesh,
    compiler_params=_sc_params,
    scratch_types=[
        pltpu.VMEM((NCH, CH), jnp.int32),
        pltpu.VMEM((CH, D), jnp.float32),
        pltpu.VMEM((ZR, D), jnp.float32),
        pltpu.VMEM_SHARED((NP, D), jnp.float32),
    ],
)
def _sc_degree(dst_hbm, out_hbm, dstv, rows, zb, acc):
    cid = lax.axis_index("c")
    sid = lax.axis_index("s")
    wid = sid * 2 + cid

    def edge_loop():
        pltpu.sync_copy(dst_hbm.at[wid], dstv)

        def orow(i, c):
            rows[i, :] = jnp.ones((D,), jnp.float32)
            return c

        lax.fori_loop(0, CH, orow, 0)

        def step(j, c):
            pltpu.sync_copy(rows, acc.at[dstv.at[j]], add=True)
            return c

        lax.fori_loop(0, NCH, step, 0)

    _sc_common(cid, sid, out_hbm, zb, acc, edge_loop)


@functools.partial(
    pl.kernel,
    out_type=jax.ShapeDtypeStruct((2, NP, D), jnp.float32),
    mesh=_mesh,
    compiler_params=_sc_params,
    scratch_types=[
        pltpu.VMEM((NCH, CH), jnp.int32),
        pltpu.VMEM((NCH, CH), jnp.int32),
        pltpu.VMEM((CH, D), jnp.float32),
        pltpu.VMEM((CH, D), jnp.float32),
        pltpu.VMEM((ZR, D), jnp.float32),
        pltpu.VMEM_SHARED((NP, D), jnp.float32),
        pltpu.SemaphoreType.DMA,
        pltpu.SemaphoreType.DMA,
    ],
)
def _sc_aggregate(src_hbm, dst_hbm, ytab_hbm, out_hbm, srcv, dstv, rows0,
                  rows1, zb, acc, sem0, sem1):
    cid = lax.axis_index("c")
    sid = lax.axis_index("s")
    wid = sid * 2 + cid

    def fire(j, buf, sem):
        pltpu.async_copy(ytab_hbm.at[srcv.at[j]], buf, sem)

    def drain(buf, sem):
        pltpu.make_async_copy(ytab_hbm.at[srcv.at[0]], buf, sem).wait()

    def scat(j, buf):
        pltpu.sync_copy(buf, acc.at[dstv.at[j]], add=True)

    def edge_loop():
        pltpu.sync_copy(src_hbm.at[wid], srcv)
        pltpu.sync_copy(dst_hbm.at[wid], dstv)
        fire(0, rows0, sem0)
        fire(1, rows1, sem1)

        def pair(j2, c):
            a = 2 * j2
            drain(rows0, sem0)
            scat(a, rows0)
            fire(a + 2, rows0, sem0)
            drain(rows1, sem1)
            scat(a + 1, rows1)
            fire(a + 3, rows1, sem1)
            return c

        lax.fori_loop(0, NCH // 2 - 1, pair, 0)
        drain(rows0, sem0)
        scat(NCH - 2, rows0)
        drain(rows1, sem1)
        scat(NCH - 1, rows1)

    _sc_common(cid, sid, out_hbm, zb, acc, edge_loop)


def _k1_body(x_ref, w1_ref, degp_ref, y1_ref, dis_ref):
    deg = degp_ref[0] + degp_ref[1] + 1.0
    dis = lax.rsqrt(deg)
    dis_ref[...] = dis
    xw = jnp.dot(x_ref[...], w1_ref[...], preferred_element_type=jnp.float32)
    y1_ref[: N, :] = xw * dis[: N, :]
    y1_ref[N:, :] = jnp.zeros((NP - N, D), jnp.float32)


_k1 = pl.pallas_call(
    _k1_body,
    out_shape=[
        jax.ShapeDtypeStruct((NP, D), jnp.float32),
        jax.ShapeDtypeStruct((NP, D), jnp.float32),
    ],
)


def _k2_body(y1_ref, dis_ref, aggp_ref, w2_ref, b1p_ref, y2_ref):
    agg = aggp_ref[0] + aggp_ref[1]
    h = jnp.maximum(dis_ref[...] * (agg + y1_ref[...]) + b1p_ref[...], 0.0)
    hw2 = jnp.dot(h, w2_ref[...], preferred_element_type=jnp.float32)
    y2_ref[...] = dis_ref[...] * hw2


_k2 = pl.pallas_call(
    _k2_body,
    out_shape=jax.ShapeDtypeStruct((NP, D), jnp.float32),
)


def _k3_body(y2_ref, dis_ref, aggp_ref, b2_ref, logits_ref, probs_ref):
    agg = aggp_ref[0, : N, :] + aggp_ref[1, : N, :]
    lg = dis_ref[: N, :] * (agg + y2_ref[: N, :]) + b2_ref[...]
    logits_ref[...] = lg
    m = jnp.max(lg, axis=1, keepdims=True)
    e = jnp.exp(lg - m)
    probs_ref[...] = e / jnp.sum(e, axis=1, keepdims=True)


_k3 = pl.pallas_call(
    _k3_body,
    out_shape=[
        jax.ShapeDtypeStruct((N, C), jnp.float32),
        jax.ShapeDtypeStruct((N, C), jnp.float32),
    ],
)


def kernel(x, edge_index, W1, b1, W2, b2):
    src = edge_index[0]
    dst = edge_index[1]
    pad = EPAD - E
    srcp = jnp.concatenate([src, jnp.zeros((pad,), jnp.int32)])
    srcp = srcp.reshape(NW, NCH, CH)
    dstp = jnp.concatenate([dst, jnp.full((pad,), NGARB, jnp.int32)])
    dstp = dstp.reshape(NW, NCH, CH)

    W1p = jnp.pad(W1, ((0, 0), (0, D - H)))
    b1p = jnp.pad(b1, (0, D - H))
    W2p = jnp.pad(W2, ((0, D - H), (0, 0)))

    degp = _sc_degree(dstp)
    y1, dis = _k1(x, W1p, degp)
    agg1 = _sc_aggregate(srcp, dstp, y1)
    y2 = _k2(y1, dis, agg1, W2p, b1p)
    agg2 = _sc_aggregate(srcp, dstp, y2)
    logits, probs = _k3(y2, dis, agg2, b2)
    return logits, probs


# speedup vs baseline: 53.3256x; 1.4754x over previous
import functools

import jax
import jax.numpy as jnp
from jax import lax
from jax.experimental import pallas as pl
from jax.experimental.pallas import tpu as pltpu
from jax.experimental.pallas import tpu_sc as plsc

N = 10000
E = 320000
F = 128
H = 4
C = 16

D = 16
NW = 32
CH = 128
NCH = 2 * (-(-E // (NW * CH * 2)))
EPW = NCH * CH
EPAD = NW * EPW
NGARB = N
ZR = 632
NP = ZR * 16

_mesh = plsc.VectorSubcoreMesh(core_axis_name="c", subcore_axis_name="s")
_sc_params = pltpu.CompilerParams(use_tc_tiling_on_sc=False)


def _sc_common(cid, sid, out_hbm, zb, acc, edge_loop):
    def zrow(i, c):
        zb[i, :] = jnp.zeros((D,), jnp.float32)
        return c

    lax.fori_loop(0, ZR, zrow, 0)
    pltpu.sync_copy(zb, acc.at[pl.ds(sid * ZR, ZR)])
    plsc.subcore_barrier()

    edge_loop()

    plsc.subcore_barrier()
    pltpu.sync_copy(acc.at[pl.ds(sid * ZR, ZR)], zb)
    pltpu.sync_copy(zb, out_hbm.at[cid, pl.ds(sid * ZR, ZR)])


@functools.partial(
    pl.kernel,
    out_type=jax.ShapeDtypeStruct((2, NP, D), jnp.float32),
    mesh=_mesh,
    compiler_params=_sc_params,
    scratch_types=[
        pltpu.VMEM((NCH, CH), jnp.int32),
        pltpu.VMEM((CH, D), jnp.float32),
        pltpu.VMEM((ZR, D), jnp.float32),
        pltpu.VMEM_SHARED((NP, D), jnp.float32),
    ],
)
def _sc_degree(dst_hbm, out_hbm, dstv, rows, zb, acc):
    cid = lax.axis_index("c")
    sid = lax.axis_index("s")
    wid = sid * 2 + cid

    def edge_loop():
        pltpu.sync_copy(dst_hbm.at[wid], dstv)

        def orow(i, c):
            rows[i, :] = jnp.ones((D,), jnp.float32)
            return c

        lax.fori_loop(0, CH, orow, 0)

        def step(j, c):
            pltpu.sync_copy(rows, acc.at[dstv.at[j]], add=True)
            return c

        lax.fori_loop(0, NCH, step, 0)

    _sc_common(cid, sid, out_hbm, zb, acc, edge_loop)


@functools.partial(
    pl.kernel,
    out_type=jax.ShapeDtypeStruct((2, NP, D), jnp.float32),
    mesh=_mesh,
    compiler_params=_sc_params,
    scratch_types=[
        pltpu.VMEM((NCH, CH), jnp.int32),
        pltpu.VMEM((NCH, CH), jnp.int32),
        pltpu.VMEM((CH, D), jnp.float32),
        pltpu.VMEM((CH, D), jnp.float32),
        pltpu.VMEM((ZR, D), jnp.float32),
        pltpu.VMEM_SHARED((NP, D), jnp.float32),
        pltpu.VMEM_SHARED((NP, D), jnp.float32),
        pltpu.SemaphoreType.DMA,
        pltpu.SemaphoreType.DMA,
    ],
)
def _sc_aggregate(src_hbm, dst_hbm, ytab_hbm, out_hbm, srcv, dstv, rows0,
                  rows1, zb, acc, ytab_sh, sem0, sem1):
    cid = lax.axis_index("c")
    sid = lax.axis_index("s")
    wid = sid * 2 + cid

    def fire(j, buf, sem):
        pltpu.async_copy(ytab_sh.at[srcv.at[j]], buf, sem)

    def drain(buf, sem):
        pltpu.make_async_copy(ytab_sh.at[srcv.at[0]], buf, sem).wait()

    def scat(j, buf):
        pltpu.sync_copy(buf, acc.at[dstv.at[j]], add=True)

    def edge_loop():
        pltpu.sync_copy(ytab_hbm.at[pl.ds(sid * ZR, ZR)], zb)
        pltpu.sync_copy(zb, ytab_sh.at[pl.ds(sid * ZR, ZR)])
        plsc.subcore_barrier()
        pltpu.sync_copy(src_hbm.at[wid], srcv)
        pltpu.sync_copy(dst_hbm.at[wid], dstv)
        fire(0, rows0, sem0)
        fire(1, rows1, sem1)

        def pair(j2, c):
            a = 2 * j2
            drain(rows0, sem0)
            scat(a, rows0)
            fire(a + 2, rows0, sem0)
            drain(rows1, sem1)
            scat(a + 1, rows1)
            fire(a + 3, rows1, sem1)
            return c

        lax.fori_loop(0, NCH // 2 - 1, pair, 0)
        drain(rows0, sem0)
        scat(NCH - 2, rows0)
        drain(rows1, sem1)
        scat(NCH - 1, rows1)

    _sc_common(cid, sid, out_hbm, zb, acc, edge_loop)


def _k1_body(x_ref, w1_ref, degp_ref, y1_ref, dis_ref):
    deg = degp_ref[0] + degp_ref[1] + 1.0
    dis = lax.rsqrt(deg)
    dis_ref[...] = dis
    xw = jnp.dot(x_ref[...], w1_ref[...], preferred_element_type=jnp.float32)
    y1_ref[: N, :] = xw * dis[: N, :]
    y1_ref[N:, :] = jnp.zeros((NP - N, D), jnp.float32)


_k1 = pl.pallas_call(
    _k1_body,
    out_shape=[
        jax.ShapeDtypeStruct((NP, D), jnp.float32),
        jax.ShapeDtypeStruct((NP, D), jnp.float32),
    ],
)


def _k2_body(y1_ref, dis_ref, aggp_ref, w2_ref, b1p_ref, y2_ref):
    agg = aggp_ref[0] + aggp_ref[1]
    h = jnp.maximum(dis_ref[...] * (agg + y1_ref[...]) + b1p_ref[...], 0.0)
    hw2 = jnp.dot(h, w2_ref[...], preferred_element_type=jnp.float32)
    y2_ref[...] = dis_ref[...] * hw2


_k2 = pl.pallas_call(
    _k2_body,
    out_shape=jax.ShapeDtypeStruct((NP, D), jnp.float32),
)


def _k3_body(y2_ref, dis_ref, aggp_ref, b2_ref, logits_ref, probs_ref):
    agg = aggp_ref[0, : N, :] + aggp_ref[1, : N, :]
    lg = dis_ref[: N, :] * (agg + y2_ref[: N, :]) + b2_ref[...]
    logits_ref[...] = lg
    m = jnp.max(lg, axis=1, keepdims=True)
    e = jnp.exp(lg - m)
    probs_ref[...] = e / jnp.sum(e, axis=1, keepdims=True)


_k3 = pl.pallas_call(
    _k3_body,
    out_shape=[
        jax.ShapeDtypeStruct((N, C), jnp.float32),
        jax.ShapeDtypeStruct((N, C), jnp.float32),
    ],
)


def kernel(x, edge_index, W1, b1, W2, b2):
    src = edge_index[0]
    dst = edge_index[1]
    pad = EPAD - E
    srcp = jnp.concatenate([src, jnp.zeros((pad,), jnp.int32)])
    srcp = srcp.reshape(NW, NCH, CH)
    dstp = jnp.concatenate([dst, jnp.full((pad,), NGARB, jnp.int32)])
    dstp = dstp.reshape(NW, NCH, CH)

    W1p = jnp.pad(W1, ((0, 0), (0, D - H)))
    b1p = jnp.pad(b1, (0, D - H))
    W2p = jnp.pad(W2, ((0, D - H), (0, 0)))

    degp = _sc_degree(dstp)
    y1, dis = _k1(x, W1p, degp)
    agg1 = _sc_aggregate(srcp, dstp, y1)
    y2 = _k2(y1, dis, agg1, W2p, b1p)
    agg2 = _sc_aggregate(srcp, dstp, y2)
    logits, probs = _k3(y2, dis, agg2, b2)
    return logits, probs
